# baseline (speedup 1.0000x reference)
import functools

import jax
import jax.numpy as jnp
from jax import lax
from jax.experimental import pallas as pl
from jax.experimental.pallas import tpu as pltpu
from jax.experimental.pallas import tpu_sc as plsc

N = 32768
B = 16
MAX_G = 8
NODE_DIM = 512
HIDDEN = 1024
NUM_DIR = 7
NSLOT = B * MAX_G

NSUB = 16
CHUNK = N // NSUB
VECS = CHUNK // 16
KCHK = 4
NBLK = VECS // KCHK
BSTEPS = 5
GROUPS_PER_CORE = 4
NEG = -1000000000.0


def _sc_body(keep_hbm, batch_hbm, node_hbm,
             selp1_hbm, rows_hbm,
             batch_v, keep_v, hist_v, allh_v, cnt_v, off_v, h16_v, pe_v,
             contrib_v, allc_v, sel128_v, sel16_v, idx_v, rows_v,
             sh_hist, sh_contrib, sem):
    c = lax.axis_index("c")
    s = lax.axis_index("s")
    base = s * CHUNK
    zeros16 = jnp.zeros((16,), jnp.int32)
    lane = lax.iota(jnp.int32, 16)

    cp_b = pltpu.async_copy(batch_hbm.at[pl.ds(base, CHUNK)], batch_v, sem)
    cp_k = pltpu.async_copy(
        keep_hbm.at[pl.ds(s * (CHUNK // 4), CHUNK // 4)], keep_v, sem)
    cp_b.wait()
    cp_k.wait()

    lane4 = lane // 4
    shifts = (lane % 4) * 8

    def kvec(o):
        w16 = plsc.load_gather(keep_v, [o // 4 + lane4])
        return (w16 >> shifts) & 1

    b_last = jnp.max(batch_v[pl.ds(CHUNK - 16, 16)])

    def blk_max(ib):
        return jnp.max(batch_v[pl.ds(((ib + 1) * KCHK - 1) * 16, 16)])

    def next_block(lo, bhi):
        def bstep(_, st):
            l, h = st
            mid = jnp.minimum((l + h) // 2, NBLK - 1)
            gt = blk_max(mid) > bhi
            return (jnp.where(gt, l, mid + 1), jnp.where(gt, mid, h))
        return lax.fori_loop(0, BSTEPS, bstep, (lo, NBLK))[0]

    hist_v[...] = zeros16

    def p1_cond(st):
        ib, done = st
        return jnp.logical_and(ib < NBLK, done == 0)

    def p1_body(st):
        ib, _ = st
        def p1_step(k, carry):
            o = (ib * KCHK + k) * 16
            b16 = batch_v[pl.ds(o, 16)]
            plsc.addupdate_scatter(hist_v, [b16], kvec(o))
            return carry
        lax.fori_loop(0, KCHK, p1_step, 0)
        bhi = blk_max(ib)
        satcur = jnp.min(jnp.where(lane == bhi, hist_v[...], 1000)) >= MAX_G
        at_end = bhi >= b_last
        done = jnp.logical_and(satcur, at_end)
        ib_next = jnp.where(jnp.logical_and(satcur, jnp.logical_not(at_end)),
                            next_block(ib + 1, bhi), ib + 1)
        return (ib_next, done.astype(jnp.int32))

    lax.while_loop(p1_cond, p1_body, (0, jnp.int32(0)))

    for j in range(NSLOT // 16):
        contrib_v[pl.ds(j * 16, 16)] = zeros16

    pltpu.sync_copy(hist_v, sh_hist.at[s])
    plsc.subcore_barrier()
    pltpu.sync_copy(sh_hist, allh_v)

    def off_step(w, acc):
        return acc + allh_v[w]

    off16 = lax.fori_loop(0, s, off_step, zeros16)
    off_v[...] = off16
    cnt_v[...] = off16

    def p2_cond(st):
        ib, done = st
        return jnp.logical_and(ib < NBLK, done == 0)

    def p2_body(st):
        ib, _ = st
        def p2_step(k, carry):
            o = (ib * KCHK + k) * 16
            b16 = batch_v[pl.ds(o, 16)]
            k16 = kvec(o)
            h16_v[...] = zeros16
            plsc.addupdate_scatter(h16_v, [b16], k16)
            h16 = h16_v[...]
            pe_v[...] = plsc.cumsum(h16) - h16
            less_b = plsc.load_gather(pe_v, [b16])
            ck_excl = plsc.cumsum(k16) - k16
            basec = plsc.load_gather(cnt_v, [b16])
            g16 = basec + ck_excl - less_b
            valid = jnp.logical_and(k16 > 0, g16 < MAX_G)
            tgt = jnp.where(valid, b16 * MAX_G + g16, 0)
            rowid = base + o + lane + 1
            plsc.store_scatter(contrib_v, [tgt], rowid, mask=valid)
            cnt_v[...] = cnt_v[...] + h16
            return carry
        lax.fori_loop(0, KCHK, p2_step, 0)
        bhi = blk_max(ib)
        dvec = jnp.logical_or(cnt_v[...] >= MAX_G,
                              cnt_v[...] - off_v[...] >= hist_v[...])
        dcur = jnp.min(jnp.where(lane == bhi, dvec.astype(jnp.int32), 1)) >= 1
        at_end = bhi >= b_last
        done = jnp.logical_and(dcur, at_end)
        ib_next = jnp.where(jnp.logical_and(dcur, jnp.logical_not(at_end)),
                            next_block(ib + 1, bhi), ib + 1)
        return (ib_next, done.astype(jnp.int32))

    lax.while_loop(p2_cond, p2_body, (0, jnp.int32(0)))

    pltpu.sync_copy(contrib_v, sh_contrib.at[s])
    plsc.subcore_barrier()

    @pl.when(s < GROUPS_PER_CORE)
    def _():
        grp = c * GROUPS_PER_CORE + s
        pltpu.sync_copy(sh_contrib, allc_v)
        for j in range(NSLOT // 16):
            def sum_step(w, acc):
                return acc + allc_v[w, pl.ds(j * 16, 16)]
            sel128_v[pl.ds(j * 16, 16)] = lax.fori_loop(
                0, NSUB, sum_step, zeros16)
        sel16 = sel128_v[pl.ds(grp * 16, 16)]
        idx_v[...] = jnp.maximum(sel16 - 1, 0)
        gat = pltpu.async_copy(node_hbm.at[idx_v], rows_v, sem)
        sel16_v[...] = sel16
        pltpu.sync_copy(sel16_v, selp1_hbm.at[pl.ds(grp * 16, 16)])
        gat.wait()
        pltpu.sync_copy(rows_v, rows_hbm.at[pl.ds(grp * 16, 16)])


@functools.partial(jax.jit, static_argnames=("interpret",))
def _sc_call(keep_i, batch_i, node_features, interpret=False):
    mesh = plsc.VectorSubcoreMesh(core_axis_name="c", subcore_axis_name="s",
                                  num_cores=2, num_subcores=NSUB)
    fn = pl.kernel(
        _sc_body,
        out_type=(
            jax.ShapeDtypeStruct((NSLOT,), jnp.int32),
            jax.ShapeDtypeStruct((NSLOT, NODE_DIM), jnp.float32),
        ),
        mesh=mesh,
        scratch_types=[
            pltpu.VMEM((CHUNK,), jnp.int32),
            pltpu.VMEM((CHUNK // 4,), jnp.int32),
            pltpu.VMEM((16,), jnp.int32),
            pltpu.VMEM((NSUB, 16), jnp.int32),
            pltpu.VMEM((16,), jnp.int32),
            pltpu.VMEM((16,), jnp.int32),
            pltpu.VMEM((16,), jnp.int32),
            pltpu.VMEM((16,), jnp.int32),
            pltpu.VMEM((NSLOT,), jnp.int32),
            pltpu.VMEM((NSUB, NSLOT), jnp.int32),
            pltpu.VMEM((NSLOT,), jnp.int32),
            pltpu.VMEM((16,), jnp.int32),
            pltpu.VMEM((16,), jnp.int32),
            pltpu.VMEM((16, NODE_DIM), jnp.float32),
            pltpu.VMEM_SHARED((NSUB, 16), jnp.int32),
            pltpu.VMEM_SHARED((NSUB, NSLOT), jnp.int32),
            pltpu.SemaphoreType.DMA,
        ],
        compiler_params=pltpu.CompilerParams(needs_layout_passes=False),
        interpret=interpret,
    )
    return fn(keep_i, batch_i, node_features)


def _tca_body(glob_ref, w1b_ref, out_ref):
    out_ref[...] = jnp.dot(glob_ref[...], w1b_ref[...],
                           preferred_element_type=jnp.float32)


@jax.jit
def _tca_call(global_features, W1):
    return pl.pallas_call(
        _tca_body,
        grid=(1,),
        out_shape=jax.ShapeDtypeStruct((B, HIDDEN), jnp.float32),
        in_specs=[
            pl.BlockSpec((B, NODE_DIM), lambda i: (0, 0)),
            pl.BlockSpec((NODE_DIM, HIDDEN), lambda i: (1, 0)),
        ],
        out_specs=pl.BlockSpec((B, HIDDEN), lambda i: (0, 0)),
    )(global_features, W1)


def _tc_body(g_ref, glob_ref, w1_ref, b1_ref, w2_ref, b2_ref, sel_ref,
             mm_ref, out_ref):
    x = g_ref[...]
    node_part = jnp.dot(x, w1_ref[...],
                        preferred_element_type=jnp.float32)
    glob_part = glob_ref[...]
    rows = lax.broadcasted_iota(jnp.int32, (NSLOT, B), 0) // MAX_G
    cols = lax.broadcasted_iota(jnp.int32, (NSLOT, B), 1)
    expand = (rows == cols).astype(jnp.float32)
    gexp = jnp.dot(expand, glob_part, preferred_element_type=jnp.float32)
    h = jnp.maximum(node_part + gexp + b1_ref[...], 0.0)
    bi = lax.broadcasted_iota(jnp.int32, (B, NSLOT), 0)
    ci = lax.broadcasted_iota(jnp.int32, (B, NSLOT), 1)
    mm = mm_ref[...]
    sel = sel_ref[...]
    pieces = []
    for g in range(MAX_G):
        sg = (ci == bi * MAX_G + g).astype(jnp.float32)
        hg = jnp.dot(sg, h, preferred_element_type=jnp.float32)
        og = jnp.dot(hg, w2_ref[...],
                     preferred_element_type=jnp.float32) + b2_ref[...]
        vg = sel[:, g:g + 1] > 0
        mg = mm[:, g * NUM_DIR:(g + 1) * NUM_DIR] > 0.0
        pieces.append(jnp.where(jnp.logical_and(vg, mg), og, NEG))
    out_ref[...] = jnp.concatenate(pieces, axis=1)


@functools.partial(jax.jit, static_argnames=("interpret",))
def _tc_call(gathered, glob_part, W1, b1, W2, b2, selp1, mm,
             interpret=False):
    return pl.pallas_call(
        _tc_body,
        grid=(1,),
        out_shape=jax.ShapeDtypeStruct((B, MAX_G * NUM_DIR), jnp.float32),
        in_specs=[
            pl.BlockSpec((NSLOT, NODE_DIM), lambda i: (0, 0)),
            pl.BlockSpec((B, HIDDEN), lambda i: (0, 0)),
            pl.BlockSpec((NODE_DIM, HIDDEN), lambda i: (0, 0)),
            pl.BlockSpec((1, HIDDEN), lambda i: (0, 0)),
            pl.BlockSpec((HIDDEN, NUM_DIR), lambda i: (0, 0)),
            pl.BlockSpec((1, NUM_DIR), lambda i: (0, 0)),
            pl.BlockSpec((B, MAX_G), lambda i: (0, 0)),
            pl.BlockSpec((B, MAX_G * NUM_DIR), lambda i: (0, 0)),
        ],
        out_specs=pl.BlockSpec((B, MAX_G * NUM_DIR), lambda i: (0, 0)),
        interpret=interpret,
    )(gathered, glob_part, W1, b1.reshape(1, HIDDEN), W2,
      b2.reshape(1, NUM_DIR), selp1.reshape(B, MAX_G),
      mm.reshape(B, MAX_G * NUM_DIR).astype(jnp.float32))


def kernel(node_features, global_features, W1, b1, W2, b2,
           group_mask_nodes, batch, maneuver_mask):
    keep_w = lax.bitcast_convert_type(
        group_mask_nodes.view(jnp.uint8).reshape(N // 4, 4), jnp.int32)
    batch_i = batch.astype(jnp.int32)
    selp1, gathered = _sc_call(keep_w, batch_i, node_features)
    glob_part = _tca_call(global_features, W1)
    return _tc_call(gathered, glob_part, W1, b1, W2, b2, selp1,
                    maneuver_mask)


# speedup vs baseline: 1.1674x; 1.1674x over previous
import functools

import jax
import jax.numpy as jnp
from jax import lax
from jax.experimental import pallas as pl
from jax.experimental.pallas import tpu as pltpu
from jax.experimental.pallas import tpu_sc as plsc

N = 32768
B = 16
MAX_G = 8
NODE_DIM = 512
HIDDEN = 1024
NUM_DIR = 7
NSLOT = B * MAX_G

NSUB = 16
CHUNK = N // NSUB
VECS = CHUNK // 16
KCHK = 4
NBLK = VECS // KCHK
BSTEPS = 5
GROUPS_PER_CORE = 4
NEG = -1000000000.0


def _sc_body(keep_hbm, batch_hbm, node_hbm,
             selp1_hbm, rows_hbm,
             batch_v, keep_v, hist_v, allh_v, cnt_v, off_v, h16_v, pe_v,
             contrib_v, allc_v, sel128_v, sel16_v, idx_v, rows_v,
             sh_hist, sh_contrib, sem):
    c = lax.axis_index("c")
    s = lax.axis_index("s")
    base = s * CHUNK
    zeros16 = jnp.zeros((16,), jnp.int32)
    lane = lax.iota(jnp.int32, 16)

    cp_b = pltpu.async_copy(batch_hbm.at[pl.ds(base, CHUNK)], batch_v, sem)
    cp_k = pltpu.async_copy(keep_hbm.at[pl.ds(base, CHUNK)], keep_v, sem)
    cp_b.wait()
    cp_k.wait()

    def kvec(o):
        return keep_v[pl.ds(o, 16)]

    b_last = jnp.max(batch_v[pl.ds(CHUNK - 16, 16)])

    def blk_max(ib):
        return jnp.max(batch_v[pl.ds(((ib + 1) * KCHK - 1) * 16, 16)])

    def next_block(lo, bhi):
        def bstep(_, st):
            l, h = st
            mid = jnp.minimum((l + h) // 2, NBLK - 1)
            gt = blk_max(mid) > bhi
            return (jnp.where(gt, l, mid + 1), jnp.where(gt, mid, h))
        return lax.fori_loop(0, BSTEPS, bstep, (lo, NBLK))[0]

    hist_v[...] = zeros16

    def p1_cond(st):
        ib, done = st
        return jnp.logical_and(ib < NBLK, done == 0)

    def p1_body(st):
        ib, _ = st
        def p1_step(k, carry):
            o = (ib * KCHK + k) * 16
            b16 = batch_v[pl.ds(o, 16)]
            plsc.addupdate_scatter(hist_v, [b16], kvec(o))
            return carry
        lax.fori_loop(0, KCHK, p1_step, 0)
        bhi = blk_max(ib)
        satcur = jnp.min(jnp.where(lane == bhi, hist_v[...], 1000)) >= MAX_G
        at_end = bhi >= b_last
        done = jnp.logical_and(satcur, at_end)
        ib_next = jnp.where(jnp.logical_and(satcur, jnp.logical_not(at_end)),
                            next_block(ib + 1, bhi), ib + 1)
        return (ib_next, done.astype(jnp.int32))

    lax.while_loop(p1_cond, p1_body, (0, jnp.int32(0)))

    for j in range(NSLOT // 16):
        contrib_v[pl.ds(j * 16, 16)] = zeros16

    pltpu.sync_copy(hist_v, sh_hist.at[s])
    plsc.subcore_barrier()
    pltpu.sync_copy(sh_hist, allh_v)

    def off_step(w, acc):
        return acc + allh_v[w]

    off16 = lax.fori_loop(0, s, off_step, zeros16)
    off_v[...] = off16
    cnt_v[...] = off16

    def p2_cond(st):
        ib, done = st
        return jnp.logical_and(ib < NBLK, done == 0)

    def p2_body(st):
        ib, _ = st
        def p2_step(k, carry):
            o = (ib * KCHK + k) * 16
            b16 = batch_v[pl.ds(o, 16)]
            k16 = kvec(o)
            h16_v[...] = zeros16
            plsc.addupdate_scatter(h16_v, [b16], k16)
            h16 = h16_v[...]
            pe_v[...] = plsc.cumsum(h16) - h16
            less_b = plsc.load_gather(pe_v, [b16])
            ck_excl = plsc.cumsum(k16) - k16
            basec = plsc.load_gather(cnt_v, [b16])
            g16 = basec + ck_excl - less_b
            valid = jnp.logical_and(k16 > 0, g16 < MAX_G)
            tgt = jnp.where(valid, b16 * MAX_G + g16, 0)
            rowid = base + o + lane + 1
            plsc.store_scatter(contrib_v, [tgt], rowid, mask=valid)
            cnt_v[...] = cnt_v[...] + h16
            return carry
        lax.fori_loop(0, KCHK, p2_step, 0)
        bhi = blk_max(ib)
        dvec = jnp.logical_or(cnt_v[...] >= MAX_G,
                              cnt_v[...] - off_v[...] >= hist_v[...])
        dcur = jnp.min(jnp.where(lane == bhi, dvec.astype(jnp.int32), 1)) >= 1
        at_end = bhi >= b_last
        done = jnp.logical_and(dcur, at_end)
        ib_next = jnp.where(jnp.logical_and(dcur, jnp.logical_not(at_end)),
                            next_block(ib + 1, bhi), ib + 1)
        return (ib_next, done.astype(jnp.int32))

    lax.while_loop(p2_cond, p2_body, (0, jnp.int32(0)))

    pltpu.sync_copy(contrib_v, sh_contrib.at[s])
    plsc.subcore_barrier()

    @pl.when(s < GROUPS_PER_CORE)
    def _():
        grp = c * GROUPS_PER_CORE + s
        pltpu.sync_copy(sh_contrib, allc_v)
        for j in range(NSLOT // 16):
            def sum_step(w, acc):
                return acc + allc_v[w, pl.ds(j * 16, 16)]
            sel128_v[pl.ds(j * 16, 16)] = lax.fori_loop(
                0, NSUB, sum_step, zeros16)
        sel16 = sel128_v[pl.ds(grp * 16, 16)]
        idx_v[...] = jnp.maximum(sel16 - 1, 0)
        gat = pltpu.async_copy(node_hbm.at[idx_v], rows_v, sem)
        sel16_v[...] = sel16
        pltpu.sync_copy(sel16_v, selp1_hbm.at[pl.ds(grp * 16, 16)])
        gat.wait()
        pltpu.sync_copy(rows_v, rows_hbm.at[pl.ds(grp * 16, 16)])


@functools.partial(jax.jit, static_argnames=("interpret",))
def _sc_call(keep_i, batch_i, node_features, interpret=False):
    mesh = plsc.VectorSubcoreMesh(core_axis_name="c", subcore_axis_name="s",
                                  num_cores=2, num_subcores=NSUB)
    fn = pl.kernel(
        _sc_body,
        out_type=(
            jax.ShapeDtypeStruct((NSLOT,), jnp.int32),
            jax.ShapeDtypeStruct((NSLOT, NODE_DIM), jnp.float32),
        ),
        mesh=mesh,
        scratch_types=[
            pltpu.VMEM((CHUNK,), jnp.int32),
            pltpu.VMEM((CHUNK,), jnp.int32),
            pltpu.VMEM((16,), jnp.int32),
            pltpu.VMEM((NSUB, 16), jnp.int32),
            pltpu.VMEM((16,), jnp.int32),
            pltpu.VMEM((16,), jnp.int32),
            pltpu.VMEM((16,), jnp.int32),
            pltpu.VMEM((16,), jnp.int32),
            pltpu.VMEM((NSLOT,), jnp.int32),
            pltpu.VMEM((NSUB, NSLOT), jnp.int32),
            pltpu.VMEM((NSLOT,), jnp.int32),
            pltpu.VMEM((16,), jnp.int32),
            pltpu.VMEM((16,), jnp.int32),
            pltpu.VMEM((16, NODE_DIM), jnp.float32),
            pltpu.VMEM_SHARED((NSUB, 16), jnp.int32),
            pltpu.VMEM_SHARED((NSUB, NSLOT), jnp.int32),
            pltpu.SemaphoreType.DMA,
        ],
        compiler_params=pltpu.CompilerParams(needs_layout_passes=False),
        interpret=interpret,
    )
    return fn(keep_i, batch_i, node_features)


def _tca_body(glob_ref, w1b_ref, out_ref):
    out_ref[...] = jnp.dot(glob_ref[...], w1b_ref[...],
                           preferred_element_type=jnp.float32)


@jax.jit
def _tca_call(global_features, W1):
    return pl.pallas_call(
        _tca_body,
        grid=(1,),
        out_shape=jax.ShapeDtypeStruct((B, HIDDEN), jnp.float32),
        in_specs=[
            pl.BlockSpec((B, NODE_DIM), lambda i: (0, 0)),
            pl.BlockSpec((NODE_DIM, HIDDEN), lambda i: (1, 0)),
        ],
        out_specs=pl.BlockSpec((B, HIDDEN), lambda i: (0, 0)),
    )(global_features, W1)


def _tc_body(g_ref, glob_ref, w1_ref, b1_ref, w2_ref, b2_ref, sel_ref,
             mm_ref, out_ref):
    x = g_ref[...]
    node_part = jnp.dot(x, w1_ref[...],
                        preferred_element_type=jnp.float32)
    glob_part = glob_ref[...]
    rows = lax.broadcasted_iota(jnp.int32, (NSLOT, B), 0) // MAX_G
    cols = lax.broadcasted_iota(jnp.int32, (NSLOT, B), 1)
    expand = (rows == cols).astype(jnp.float32)
    gexp = jnp.dot(expand, glob_part, preferred_element_type=jnp.float32)
    h = jnp.maximum(node_part + gexp + b1_ref[...], 0.0)
    bi = lax.broadcasted_iota(jnp.int32, (B, NSLOT), 0)
    ci = lax.broadcasted_iota(jnp.int32, (B, NSLOT), 1)
    pieces = []
    for g in range(MAX_G):
        sg = (ci == bi * MAX_G + g).astype(jnp.float32)
        hg = jnp.dot(sg, h, preferred_element_type=jnp.float32)
        og = jnp.dot(hg, w2_ref[...],
                     preferred_element_type=jnp.float32) + b2_ref[...]
        pieces.append(og)
    out56 = jnp.concatenate(pieces, axis=1)
    selmat = jnp.where(ci // MAX_G == bi,
                       jnp.broadcast_to(sel_ref[...], (B, NSLOT)),
                       0).astype(jnp.float32)
    si = lax.broadcasted_iota(jnp.int32, (NSLOT, MAX_G * NUM_DIR), 0)
    oi = lax.broadcasted_iota(jnp.int32, (NSLOT, MAX_G * NUM_DIR), 1)
    proj = (oi // NUM_DIR == si % MAX_G).astype(jnp.float32)
    selv = jnp.dot(selmat, proj, preferred_element_type=jnp.float32)
    keepmask = jnp.logical_and(selv > 0.0, mm_ref[...] > 0.0)
    out_ref[...] = jnp.where(keepmask, out56, NEG)


@functools.partial(jax.jit, static_argnames=("interpret",))
def _tc_call(gathered, glob_part, W1, b1, W2, b2, selp1, mm,
             interpret=False):
    return pl.pallas_call(
        _tc_body,
        grid=(1,),
        out_shape=jax.ShapeDtypeStruct((B, MAX_G * NUM_DIR), jnp.float32),
        in_specs=[
            pl.BlockSpec((NSLOT, NODE_DIM), lambda i: (0, 0)),
            pl.BlockSpec((B, HIDDEN), lambda i: (0, 0)),
            pl.BlockSpec((NODE_DIM, HIDDEN), lambda i: (0, 0)),
            pl.BlockSpec((1, HIDDEN), lambda i: (0, 0)),
            pl.BlockSpec((HIDDEN, NUM_DIR), lambda i: (0, 0)),
            pl.BlockSpec((1, NUM_DIR), lambda i: (0, 0)),
            pl.BlockSpec((1, NSLOT), lambda i: (0, 0)),
            pl.BlockSpec((B, MAX_G * NUM_DIR), lambda i: (0, 0)),
        ],
        out_specs=pl.BlockSpec((B, MAX_G * NUM_DIR), lambda i: (0, 0)),
        interpret=interpret,
    )(gathered, glob_part, W1, b1.reshape(1, HIDDEN), W2,
      b2.reshape(1, NUM_DIR), selp1.reshape(1, NSLOT),
      mm.reshape(B, MAX_G * NUM_DIR).astype(jnp.float32))


def kernel(node_features, global_features, W1, b1, W2, b2,
           group_mask_nodes, batch, maneuver_mask):
    keep_i = group_mask_nodes.astype(jnp.int32)
    batch_i = batch.astype(jnp.int32)
    selp1, gathered = _sc_call(keep_i, batch_i, node_features)
    glob_part = _tca_call(global_features, W1)
    return _tc_call(gathered, glob_part, W1, b1, W2, b2, selp1,
                    maneuver_mask)
